# Initial kernel scaffold; baseline (speedup 1.0000x reference)
import jax
import jax.numpy as jnp
from jax.experimental import pallas as pl


def kernel(scores):
    raise NotImplementedError("write your pallas kernel here")


# speedup vs baseline: 1311.5244x; 1311.5244x over previous
import jax
import jax.numpy as jnp
from jax.experimental import pallas as pl
from jax.experimental.pallas import tpu as pltpu

_TAU = 0.5
_EPS = 1e-10


def _ccl_body(x_ref, out_ref, colsum_ref, acc_ref):
    phase = pl.program_id(0)
    step = pl.program_id(1)
    nsteps = pl.num_programs(1)
    b, n = x_ref.shape

    s = jnp.exp(x_ref[...] * (1.0 / _TAU))

    @pl.when(phase == 0)
    def _sums():
        @pl.when(step == 0)
        def _init():
            colsum_ref[...] = jnp.zeros_like(colsum_ref)
            acc_ref[...] = jnp.zeros_like(acc_ref)

        colsum_ref[...] += s.sum(axis=0, keepdims=True)

    @pl.when(phase == 1)
    def _loss():
        rsum = s.sum(axis=1, keepdims=True) + _EPS
        csum = colsum_ref[...] + _EPS
        term = jnp.log((1.0 - s / rsum + _EPS) * (1.0 - s / csum + _EPS))
        rows = step * b + jax.lax.broadcasted_iota(jnp.int32, (b, n), 0)
        cols = jax.lax.broadcasted_iota(jnp.int32, (b, n), 1)
        term = jnp.where(rows == cols, 0.0, term)
        acc_ref[...] += term.sum(axis=1, keepdims=True).sum(axis=0, keepdims=True)

        @pl.when(step == nsteps - 1)
        def _finish():
            out_ref[...] = acc_ref[...] * (-1.0 / n)


def kernel(scores):
    n = scores.shape[0]
    block = 512
    grid = (2, n // block)
    out = pl.pallas_call(
        _ccl_body,
        grid=grid,
        in_specs=[pl.BlockSpec((block, n), lambda p, i: (i, 0))],
        out_specs=pl.BlockSpec((1, 1), lambda p, i: (0, 0)),
        out_shape=jax.ShapeDtypeStruct((1, 1), jnp.float32),
        scratch_shapes=[
            pltpu.VMEM((1, n), jnp.float32),
            pltpu.VMEM((1, 1), jnp.float32),
        ],
        compiler_params=pltpu.CompilerParams(
            dimension_semantics=("arbitrary", "arbitrary"),
        ),
    )(scores)
    return out[0, 0]


# speedup vs baseline: 1476.6892x; 1.1259x over previous
import jax
import jax.numpy as jnp
from jax.experimental import pallas as pl
from jax.experimental.pallas import tpu as pltpu

_TAU = 0.5
_EPS = 1e-10


def _ccl_body(x_ref, out_ref, colsum_ref, acc_ref, cache_ref):
    phase = pl.program_id(0)
    step = pl.program_id(1)
    nsteps = pl.num_programs(1)
    b = cache_ref.shape[0] // nsteps
    n = cache_ref.shape[1]

    @pl.when(phase == 0)
    def _sums():
        @pl.when(step == 0)
        def _init():
            colsum_ref[...] = jnp.zeros_like(colsum_ref)
            acc_ref[...] = jnp.zeros_like(acc_ref)

        s = jnp.exp(x_ref[...] * (1.0 / _TAU))
        colsum_ref[...] += s.sum(axis=0, keepdims=True)
        cache_ref[pl.ds(step * b, b), :] = s.astype(jnp.bfloat16)

    @pl.when(phase == 1)
    def _loss():
        s = cache_ref[pl.ds(step * b, b), :].astype(jnp.float32)
        rsum = s.sum(axis=1, keepdims=True) + _EPS
        csum = colsum_ref[...] + _EPS
        term = jnp.log((1.0 - s / rsum + _EPS) * (1.0 - s / csum + _EPS))
        rows = step * b + jax.lax.broadcasted_iota(jnp.int32, (b, n), 0)
        cols = jax.lax.broadcasted_iota(jnp.int32, (b, n), 1)
        term = jnp.where(rows == cols, 0.0, term)
        acc_ref[...] += term.sum(axis=1, keepdims=True).sum(axis=0, keepdims=True)

        @pl.when(step == nsteps - 1)
        def _finish():
            out_ref[...] = acc_ref[...] * (-1.0 / n)


def kernel(scores):
    n = scores.shape[0]
    block = 512
    nsteps = n // block
    grid = (2, nsteps)
    out = pl.pallas_call(
        _ccl_body,
        grid=grid,
        in_specs=[pl.BlockSpec(
            (block, n),
            lambda p, i: (jnp.where(p == 0, i, nsteps - 1), 0))],
        out_specs=pl.BlockSpec((1, 1), lambda p, i: (0, 0)),
        out_shape=jax.ShapeDtypeStruct((1, 1), jnp.float32),
        scratch_shapes=[
            pltpu.VMEM((1, n), jnp.float32),
            pltpu.VMEM((1, 1), jnp.float32),
            pltpu.VMEM((n, n), jnp.bfloat16),
        ],
        compiler_params=pltpu.CompilerParams(
            dimension_semantics=("arbitrary", "arbitrary"),
        ),
    )(scores)
    return out[0, 0]
